# baseline (speedup 1.0000x reference)
import functools
import math

import numpy as np
import jax
import jax.numpy as jnp
from jax import lax
from jax.experimental import pallas as pl
from jax.experimental.pallas import tpu as pltpu
from jax.experimental.pallas import tpu_sc as plsc

_L = 16


def _pe_table(seq_len, d_model):
    position = np.arange(seq_len, dtype=np.float32)[:, None]
    div_term = np.exp(
        np.arange(0, d_model, 2, dtype=np.float32) * (-math.log(10000.0) / d_model)
    )
    pe = np.zeros((seq_len, d_model), dtype=np.float32)
    pe[:, 0::2] = np.sin(position * div_term)
    pe[:, 1::2] = np.cos(position * div_term)
    return jnp.asarray(pe)


def _lane_allsum(x):
    ii = lax.iota(jnp.int32, _L)
    for k in (8, 4, 2, 1):
        x = x + x.at[ii ^ k].get(mode="promise_in_bounds")
    return x


def _rsqrt_vec(x):
    i = lax.bitcast_convert_type(x, jnp.int32)
    i = jnp.int32(0x5F3759DF) - lax.shift_right_logical(i, 1)
    y = lax.bitcast_convert_type(i, jnp.float32)
    for _ in range(3):
        y = y * (jnp.float32(1.5) - jnp.float32(0.5) * x * y * y)
    return y


@functools.lru_cache(maxsize=None)
def _make_sc_kernel(B, S, D, C):
    info = plsc.get_sparse_core_info()
    NC, NS = info.num_cores, info.num_subcores
    NW = NC * NS
    T = B * S
    TPW = T // NW
    NCH = TPW // C
    JD = D // _L
    assert T % NW == 0 and TPW % C == 0 and D % _L == 0

    mesh = plsc.VectorSubcoreMesh(core_axis_name="c", subcore_axis_name="s")

    @functools.partial(
        pl.kernel,
        mesh=mesh,
        out_type=jax.ShapeDtypeStruct((T, D), jnp.float32),
        scratch_types=[
            pltpu.VMEM((TPW,), jnp.int32),
            pltpu.VMEM((C, D), jnp.float32),
            pltpu.VMEM((C, D), jnp.float32),
            pltpu.VMEM((D,), jnp.float32),
            pltpu.VMEM((D,), jnp.float32),
            pltpu.SemaphoreType.DMA,
        ],
    )
    def k(ids_hbm, pe_hbm, table_hbm, gamma_hbm, beta_hbm, out_hbm,
          idx_v, rows_v, pe_v, gamma_v, beta_v, sem):
        w = lax.axis_index("s") * NC + lax.axis_index("c")
        base = w * TPW
        pos0 = lax.rem(base, S)
        pltpu.sync_copy(ids_hbm.at[pl.ds(base, TPW)], idx_v)
        pltpu.sync_copy(gamma_hbm, gamma_v)
        pltpu.sync_copy(beta_hbm, beta_v)

        def chunk(g, _):
            tok0 = base + g * C
            cp = pltpu.async_copy(
                table_hbm.at[idx_v.at[pl.ds(g * C, C)]], rows_v, sem)
            pltpu.sync_copy(pe_hbm.at[pl.ds(pos0 + g * C, C)], pe_v)
            cp.wait()

            def row(r, _r):
                def acc(j, carry):
                    s, ss = carry
                    sl = pl.ds(j * _L, _L)
                    x = rows_v[r, sl] + pe_v[r, sl]
                    rows_v[r, sl] = x
                    return s + x, ss + x * x

                z = jnp.zeros((_L,), jnp.float32)
                s, ss = lax.fori_loop(0, JD, acc, (z, z))
                mv = _lane_allsum(s) * jnp.float32(1.0 / D)
                var = _lane_allsum(ss) * jnp.float32(1.0 / D) - mv * mv
                rstd = _rsqrt_vec(var + jnp.float32(1e-5))

                def norm(j, _n):
                    sl = pl.ds(j * _L, _L)
                    x = rows_v[r, sl]
                    rows_v[r, sl] = (x - mv) * rstd * gamma_v[sl] + beta_v[sl]
                    return 0

                lax.fori_loop(0, JD, norm, 0)
                return 0

            lax.fori_loop(0, C, row, 0)
            pltpu.sync_copy(rows_v, out_hbm.at[pl.ds(tok0, C)])
            return 0

        lax.fori_loop(0, NCH, chunk, 0)

    return k


def kernel(token_ids, token_types, table, gamma, beta):
    B, S = token_ids.shape
    _, D = table.shape
    ids = token_ids.reshape(-1).astype(jnp.int32)
    pe = _pe_table(S, D)
    out = _make_sc_kernel(B, S, D, 16)(ids, pe, table, gamma, beta)
    return out.reshape(B, S, D)
